# Initial kernel scaffold; baseline (speedup 1.0000x reference)
import jax
import jax.numpy as jnp
from jax.experimental import pallas as pl


def kernel(x):
    raise NotImplementedError("write your pallas kernel here")


# speedup vs baseline: 3.0869x; 3.0869x over previous
import functools

import jax
import jax.numpy as jnp
from jax import lax
from jax.experimental import pallas as pl
from jax.experimental.pallas import tpu as pltpu
from jax.experimental.pallas import tpu_sc as plsc

_VOXEL_SIZE = 0.05
_NUM_DISTS = 512
_GRID = 21
_V = _GRID ** 3
_VPAD = 9264
_NUM_CH = 10
_LANES = 16


def _bin_body(xt_ref, spt_ref, flat_ref, nb_ref):
    pts = xt_ref[0]
    mn = jnp.min(pts, axis=1, keepdims=True)
    vox = jnp.clip(jnp.floor((pts - mn) / _VOXEL_SIZE).astype(jnp.int32),
                   0, _GRID - 1)
    flat_ref[0] = (vox[0:1] * (_GRID * _GRID) + vox[1:2] * _GRID + vox[2:3])
    sp = spt_ref[0]
    svox = jnp.clip(jnp.floor((sp - mn) / _VOXEL_SIZE).astype(jnp.int32),
                    0, _GRID - 1)
    nb_ref[0] = (svox[0:1] * (_GRID * _GRID) + svox[1:2] * _GRID + svox[2:3])


def _fin_body(p_ref, o_ref):
    p = p_ref[0]
    cnt = jnp.maximum(p[0:1], 1.0)
    m0 = p[1:2] / cnt
    m1 = p[2:3] / cnt
    m2 = p[3:4] / cnt
    c00 = p[4:5] / cnt - m0 * m0
    c01 = p[5:6] / cnt - m0 * m1
    c02 = p[6:7] / cnt - m0 * m2
    c11 = p[7:8] / cnt - m1 * m1
    c12 = p[8:9] / cnt - m1 * m2
    c22 = p[9:10] / cnt - m2 * m2
    o_ref[0] = jnp.concatenate(
        [m0, m1, m2, c00, c01, c02, c01, c11, c12, c02, c12, c22], axis=0)


def _sc_body(ntask, xt_hbm, flat_hbm, nb_hbm, out_hbm,
             flat_v, ra_v, rb_v, table_v, nb_v, row_v):
    n = flat_v.shape[0]
    k = nb_v.shape[0]
    nc = 2
    wid = lax.axis_index("s") * nc + lax.axis_index("c")
    nw = 32
    for slot in range((ntask + nw - 1) // nw):
        t = wid + slot * nw

        @pl.when(t < ntask)
        def _task():
            b = ((t >= 10).astype(jnp.int32) + (t >= 20).astype(jnp.int32)
                 + (t >= 30).astype(jnp.int32))
            c = t - 10 * b
            d1 = jnp.where(c < 4, jnp.maximum(c - 1, 0),
                           jnp.where(c < 7, 0, jnp.where(c < 9, 1, 2)))
            d2 = jnp.where(c < 4, 0,
                           jnp.where(c < 7, c - 4, jnp.where(c < 9, c - 6, 2)))
            pltpu.sync_copy(flat_hbm.at[b, 0], flat_v)
            pltpu.sync_copy(xt_hbm.at[b, d1], ra_v)
            pltpu.sync_copy(xt_hbm.at[b, d2], rb_v)

            def zb(i, carry):
                table_v[pl.ds(i * _LANES, _LANES)] = jnp.zeros(
                    (_LANES,), jnp.float32)
                return carry
            lax.fori_loop(0, _VPAD // _LANES, zb, 0)

            is_count = c == 0
            is_linear = c < 4

            def sb(i, carry):
                s = pl.ds(i * _LANES, _LANES)
                idx = flat_v[s]
                a = ra_v[s]
                bb = rb_v[s]
                val = jnp.where(is_linear, a, a * bb)
                val = jnp.where(is_count,
                                jnp.ones((_LANES,), jnp.float32), val)
                plsc.addupdate_scatter(table_v, [idx], val)
                return carry
            lax.fori_loop(0, n // _LANES, sb, 0)

            pltpu.sync_copy(nb_hbm.at[b, 0], nb_v)

            def gb(i, carry):
                s = pl.ds(i * _LANES, _LANES)
                row_v[s] = plsc.load_gather(table_v, [nb_v[s]])
                return carry
            lax.fori_loop(0, k // _LANES, gb, 0)

            pltpu.sync_copy(row_v, out_hbm.at[b, c])


def kernel(x):
    B, N, _ = x.shape
    K = _NUM_DISTS

    skeys = jax.random.split(jax.random.key(42), B)
    sampled_idx = jax.vmap(
        lambda kk: jax.random.permutation(kk, N)[:K])(skeys)
    sampled_pcd = jnp.take_along_axis(x, sampled_idx[..., None], axis=1)

    xt = jnp.transpose(x, (0, 2, 1))
    spt = jnp.transpose(sampled_pcd, (0, 2, 1))

    flat, nb = pl.pallas_call(
        _bin_body,
        grid=(B,),
        in_specs=[
            pl.BlockSpec((1, 3, N), lambda b: (b, 0, 0)),
            pl.BlockSpec((1, 3, K), lambda b: (b, 0, 0)),
        ],
        out_specs=[
            pl.BlockSpec((1, 1, N), lambda b: (b, 0, 0)),
            pl.BlockSpec((1, 1, K), lambda b: (b, 0, 0)),
        ],
        out_shape=[
            jax.ShapeDtypeStruct((B, 1, N), jnp.int32),
            jax.ShapeDtypeStruct((B, 1, K), jnp.int32),
        ],
    )(xt, spt)

    ntask = B * _NUM_CH
    mesh = plsc.VectorSubcoreMesh(core_axis_name="c", subcore_axis_name="s",
                                  num_cores=2, num_subcores=16)
    partial = pl.kernel(
        functools.partial(_sc_body, ntask),
        out_type=jax.ShapeDtypeStruct((B, _NUM_CH, K), jnp.float32),
        mesh=mesh,
        compiler_params=pltpu.CompilerParams(needs_layout_passes=False),
        scratch_types=[
            pltpu.VMEM((N,), jnp.int32),
            pltpu.VMEM((N,), jnp.float32),
            pltpu.VMEM((N,), jnp.float32),
            pltpu.VMEM((_VPAD,), jnp.float32),
            pltpu.VMEM((K,), jnp.int32),
            pltpu.VMEM((K,), jnp.float32),
        ],
    )(xt, flat, nb)

    out12 = pl.pallas_call(
        _fin_body,
        grid=(B,),
        in_specs=[pl.BlockSpec((1, _NUM_CH, K), lambda b: (b, 0, 0))],
        out_specs=pl.BlockSpec((1, 12, K), lambda b: (b, 0, 0)),
        out_shape=jax.ShapeDtypeStruct((B, 12, K), jnp.float32),
    )(partial)

    return jnp.transpose(out12, (0, 2, 1))


# speedup vs baseline: 8.6432x; 2.8000x over previous
import functools

import jax
import jax.numpy as jnp
from jax import lax
from jax.experimental import pallas as pl
from jax.experimental.pallas import tpu as pltpu
from jax.experimental.pallas import tpu_sc as plsc

_VOXEL_SIZE = 0.05
_NUM_DISTS = 512
_GRID = 21
_V = _GRID ** 3
_VPAD = 9264
_NUM_CH = 10
_LANES = 16


def _bin_body(xt_ref, spt_ref, flat_ref, nb_ref):
    pts = xt_ref[0]
    mn = jnp.min(pts, axis=1, keepdims=True)
    vox = jnp.clip(jnp.floor((pts - mn) / _VOXEL_SIZE).astype(jnp.int32),
                   0, _GRID - 1)
    flat_ref[0] = (vox[0:1] * (_GRID * _GRID) + vox[1:2] * _GRID + vox[2:3])
    sp = spt_ref[0]
    svox = jnp.clip(jnp.floor((sp - mn) / _VOXEL_SIZE).astype(jnp.int32),
                    0, _GRID - 1)
    nb_ref[0] = (svox[0:1] * (_GRID * _GRID) + svox[1:2] * _GRID + svox[2:3])


def _fin_body(p_ref, o_ref):
    p = p_ref[0]
    cnt = jnp.maximum(p[0:1], 1.0)
    m0 = p[1:2] / cnt
    m1 = p[2:3] / cnt
    m2 = p[3:4] / cnt
    c00 = p[4:5] / cnt - m0 * m0
    c01 = p[5:6] / cnt - m0 * m1
    c02 = p[6:7] / cnt - m0 * m2
    c11 = p[7:8] / cnt - m1 * m1
    c12 = p[8:9] / cnt - m1 * m2
    c22 = p[9:10] / cnt - m2 * m2
    o_ref[0] = jnp.concatenate(
        [m0, m1, m2, c00, c01, c02, c01, c11, c12, c02, c12, c22], axis=0)


def _sc_body(ntask, xt_hbm, flat_hbm, nb_hbm, out_hbm,
             flat_v, ra_v, rb_v, table_v, nb_v, row_v):
    n = flat_v.shape[0]
    k = nb_v.shape[0]
    nc = 2
    wid = lax.axis_index("s") * nc + lax.axis_index("c")
    nw = 32
    for slot in range((ntask + nw - 1) // nw):
        t = wid + slot * nw

        @pl.when(t < ntask)
        def _task():
            b = ((t >= 10).astype(jnp.int32) + (t >= 20).astype(jnp.int32)
                 + (t >= 30).astype(jnp.int32))
            c = t - 10 * b
            d1 = jnp.where(c < 4, jnp.maximum(c - 1, 0),
                           jnp.where(c < 7, 0, jnp.where(c < 9, 1, 2)))
            d2 = jnp.where(c < 4, 0,
                           jnp.where(c < 7, c - 4, jnp.where(c < 9, c - 6, 2)))
            pltpu.sync_copy(flat_hbm.at[b, 0], flat_v)
            pltpu.sync_copy(xt_hbm.at[b, d1], ra_v)
            pltpu.sync_copy(xt_hbm.at[b, d2], rb_v)

            def zb(i, carry):
                table_v[pl.ds(i * _LANES, _LANES)] = jnp.zeros(
                    (_LANES,), jnp.float32)
                return carry
            lax.fori_loop(0, _VPAD // _LANES, zb, 0)

            is_count = c == 0
            is_linear = c < 4

            def sb(i, carry):
                s = pl.ds(i * _LANES, _LANES)
                idx = flat_v[s]
                a = ra_v[s]
                bb = rb_v[s]
                val = jnp.where(is_linear, a, a * bb)
                val = jnp.where(is_count,
                                jnp.ones((_LANES,), jnp.float32), val)
                plsc.addupdate_scatter(table_v, [idx], val)
                return carry
            lax.fori_loop(0, n // _LANES, sb, 0)

            pltpu.sync_copy(nb_hbm.at[b, 0], nb_v)

            def gb(i, carry):
                s = pl.ds(i * _LANES, _LANES)
                row_v[s] = plsc.load_gather(table_v, [nb_v[s]])
                return carry
            lax.fori_loop(0, k // _LANES, gb, 0)

            pltpu.sync_copy(row_v, out_hbm.at[b, c])


def kernel(x):
    B, N, _ = x.shape
    K = _NUM_DISTS

    with jax.ensure_compile_time_eval():
        skeys = jax.random.split(jax.random.key(42), B)
        sampled_idx = jax.vmap(
            lambda kk: jax.random.permutation(kk, N)[:K])(skeys)
    sampled_pcd = jnp.take_along_axis(x, sampled_idx[..., None], axis=1)

    xt = jnp.transpose(x, (0, 2, 1))
    spt = jnp.transpose(sampled_pcd, (0, 2, 1))

    flat, nb = pl.pallas_call(
        _bin_body,
        grid=(B,),
        in_specs=[
            pl.BlockSpec((1, 3, N), lambda b: (b, 0, 0)),
            pl.BlockSpec((1, 3, K), lambda b: (b, 0, 0)),
        ],
        out_specs=[
            pl.BlockSpec((1, 1, N), lambda b: (b, 0, 0)),
            pl.BlockSpec((1, 1, K), lambda b: (b, 0, 0)),
        ],
        out_shape=[
            jax.ShapeDtypeStruct((B, 1, N), jnp.int32),
            jax.ShapeDtypeStruct((B, 1, K), jnp.int32),
        ],
    )(xt, spt)

    ntask = B * _NUM_CH
    mesh = plsc.VectorSubcoreMesh(core_axis_name="c", subcore_axis_name="s",
                                  num_cores=2, num_subcores=16)
    partial = pl.kernel(
        functools.partial(_sc_body, ntask),
        out_type=jax.ShapeDtypeStruct((B, _NUM_CH, K), jnp.float32),
        mesh=mesh,
        compiler_params=pltpu.CompilerParams(needs_layout_passes=False),
        scratch_types=[
            pltpu.VMEM((N,), jnp.int32),
            pltpu.VMEM((N,), jnp.float32),
            pltpu.VMEM((N,), jnp.float32),
            pltpu.VMEM((_VPAD,), jnp.float32),
            pltpu.VMEM((K,), jnp.int32),
            pltpu.VMEM((K,), jnp.float32),
        ],
    )(xt, flat, nb)

    out12 = pl.pallas_call(
        _fin_body,
        grid=(B,),
        in_specs=[pl.BlockSpec((1, _NUM_CH, K), lambda b: (b, 0, 0))],
        out_specs=pl.BlockSpec((1, 12, K), lambda b: (b, 0, 0)),
        out_shape=jax.ShapeDtypeStruct((B, 12, K), jnp.float32),
    )(partial)

    return jnp.transpose(out12, (0, 2, 1))


# speedup vs baseline: 13.3201x; 1.5411x over previous
import functools

import jax
import jax.numpy as jnp
from jax import lax
from jax.experimental import pallas as pl
from jax.experimental.pallas import tpu as pltpu
from jax.experimental.pallas import tpu_sc as plsc

_VOXEL_SIZE = 0.05
_NUM_DISTS = 512
_GRID = 21
_V = _GRID ** 3
_VPAD = 9264
_NUM_CH = 10
_LANES = 16


def _bin_body(xt_ref, flat_ref):
    pts = xt_ref[0]
    mn = jnp.min(pts, axis=1, keepdims=True)
    vox = jnp.clip(jnp.floor((pts - mn) / _VOXEL_SIZE).astype(jnp.int32),
                   0, _GRID - 1)
    flat_ref[0] = (vox[0:1] * (_GRID * _GRID) + vox[1:2] * _GRID + vox[2:3])


def _fin_body(p_ref, o_ref):
    p = p_ref[0]
    cnt = jnp.maximum(p[0:1], 1.0)
    m0 = p[1:2] / cnt
    m1 = p[2:3] / cnt
    m2 = p[3:4] / cnt
    c00 = p[4:5] / cnt - m0 * m0
    c01 = p[5:6] / cnt - m0 * m1
    c02 = p[6:7] / cnt - m0 * m2
    c11 = p[7:8] / cnt - m1 * m1
    c12 = p[8:9] / cnt - m1 * m2
    c22 = p[9:10] / cnt - m2 * m2
    o_ref[0] = jnp.concatenate(
        [m0, m1, m2, c00, c01, c02, c01, c11, c12, c02, c12, c22], axis=0)


_PAIR_ROWS = [(0, 0, 0), (1, 2, 0), (0, 1, 0), (0, 2, 1), (1, 2, 0)]
_PAIR_VALS = [
    (None, "a"), ("a", "b"), ("aa", "ab"), ("ab", "cc"), ("ab", "bb")]
_UNROLL = 4


def _sc_body(nbatch, xt_hbm, flat_hbm, sidx_hbm, zeros_hbm, out_hbm,
             flat_v, ra_v, rb_v, rc_v, t0_v, t1_v, sidx_v, o0_v, o1_v):
    n = flat_v.shape[0]
    k = sidx_v.shape[0]
    wid = lax.axis_index("s") * 2 + lax.axis_index("c")

    for t in range(nbatch * 5):
        b, p = t // 5, t % 5
        da, db, dc = _PAIR_ROWS[p]
        e0, e1 = _PAIR_VALS[p]

        @pl.when(wid == t)
        def _task(b=b, p=p, da=da, db=db, dc=dc, e0=e0, e1=e1):
            pltpu.sync_copy(flat_hbm.at[b, 0], flat_v)
            pltpu.sync_copy(xt_hbm.at[b, da], ra_v)
            pltpu.sync_copy(xt_hbm.at[b, db], rb_v)
            pltpu.sync_copy(xt_hbm.at[b, dc], rc_v)
            pltpu.sync_copy(zeros_hbm, t0_v)
            pltpu.sync_copy(zeros_hbm, t1_v)
            pltpu.sync_copy(sidx_hbm.at[b], sidx_v)

            def val(expr, a, bb, cc):
                if expr is None:
                    return jnp.ones((_LANES,), jnp.float32)
                if expr == "a":
                    return a
                if expr == "b":
                    return bb
                if expr == "aa":
                    return a * a
                if expr == "ab":
                    return a * bb
                if expr == "bb":
                    return bb * bb
                return cc * cc

            def sb(i, carry):
                for u in range(_UNROLL):
                    s = pl.ds((i * _UNROLL + u) * _LANES, _LANES)
                    idx = flat_v[s]
                    a = ra_v[s]
                    bb = rb_v[s]
                    cc = rc_v[s]
                    plsc.addupdate_scatter(t0_v, [idx], val(e0, a, bb, cc))
                    plsc.addupdate_scatter(t1_v, [idx], val(e1, a, bb, cc))
                return carry
            lax.fori_loop(0, n // (_LANES * _UNROLL), sb, 0)

            def gb(i, carry):
                s = pl.ds(i * _LANES, _LANES)
                nb = plsc.load_gather(flat_v, [sidx_v[s]])
                o0_v[s] = plsc.load_gather(t0_v, [nb])
                o1_v[s] = plsc.load_gather(t1_v, [nb])
                return carry
            lax.fori_loop(0, k // _LANES, gb, 0)

            pltpu.sync_copy(o0_v, out_hbm.at[b, 2 * p])
            pltpu.sync_copy(o1_v, out_hbm.at[b, 2 * p + 1])


def kernel(x):
    B, N, _ = x.shape
    K = _NUM_DISTS

    def _make_sidx():
        skeys = jax.random.split(jax.random.key(42), B)
        return jax.vmap(
            lambda kk: jax.random.permutation(kk, N)[:K])(skeys)

    try:
        with jax.ensure_compile_time_eval():
            sampled_idx = _make_sidx()
    except Exception:
        sampled_idx = _make_sidx()
    sidx = sampled_idx.astype(jnp.int32)
    zeros = jnp.zeros((_VPAD,), jnp.float32)

    xt = jnp.transpose(x, (0, 2, 1))

    flat = pl.pallas_call(
        _bin_body,
        grid=(B,),
        in_specs=[pl.BlockSpec((1, 3, N), lambda b: (b, 0, 0))],
        out_specs=pl.BlockSpec((1, 1, N), lambda b: (b, 0, 0)),
        out_shape=jax.ShapeDtypeStruct((B, 1, N), jnp.int32),
    )(xt)

    mesh = plsc.VectorSubcoreMesh(core_axis_name="c", subcore_axis_name="s",
                                  num_cores=2, num_subcores=16)
    partial = pl.kernel(
        functools.partial(_sc_body, B),
        out_type=jax.ShapeDtypeStruct((B, _NUM_CH, K), jnp.float32),
        mesh=mesh,
        compiler_params=pltpu.CompilerParams(needs_layout_passes=False,
                                             use_tc_tiling_on_sc=False),
        scratch_types=[
            pltpu.VMEM((N,), jnp.int32),
            pltpu.VMEM((N,), jnp.float32),
            pltpu.VMEM((N,), jnp.float32),
            pltpu.VMEM((N,), jnp.float32),
            pltpu.VMEM((_VPAD,), jnp.float32),
            pltpu.VMEM((_VPAD,), jnp.float32),
            pltpu.VMEM((K,), jnp.int32),
            pltpu.VMEM((K,), jnp.float32),
            pltpu.VMEM((K,), jnp.float32),
        ],
    )(xt, flat, sidx, zeros)

    out12 = pl.pallas_call(
        _fin_body,
        grid=(B,),
        in_specs=[pl.BlockSpec((1, _NUM_CH, K), lambda b: (b, 0, 0))],
        out_specs=pl.BlockSpec((1, 12, K), lambda b: (b, 0, 0)),
        out_shape=jax.ShapeDtypeStruct((B, 12, K), jnp.float32),
    )(partial)

    return jnp.transpose(out12, (0, 2, 1))
